# Initial kernel scaffold; baseline (speedup 1.0000x reference)
import jax
import jax.numpy as jnp
from jax.experimental import pallas as pl


def kernel(hidden_states, scores, attention_mask, keep_k):
    raise NotImplementedError("write your pallas kernel here")


# speedup vs baseline: 1.4219x; 1.4219x over previous
import functools

import jax
import jax.numpy as jnp
from jax import lax
from jax.experimental import pallas as pl
from jax.experimental.pallas import tpu as pltpu
from jax.experimental.pallas import tpu_sc as plsc

L = 16


def _make_select(B, S, K):
    NV = S // L
    mesh = plsc.VectorSubcoreMesh(core_axis_name="c", subcore_axis_name="s")
    NC = mesh.num_cores

    @functools.partial(
        pl.kernel,
        out_type=jax.ShapeDtypeStruct((B, K), jnp.int32),
        mesh=mesh,
        compiler_params=pltpu.CompilerParams(needs_layout_passes=False),
        scratch_types=[
            pltpu.VMEM((S,), jnp.uint32),
            pltpu.VMEM((256,), jnp.int32),
            pltpu.VMEM((K,), jnp.int32),
        ],
    )
    def select(keys_hbm, idx_hbm, keys_v, hist_v, idx_v):
        wid = lax.axis_index("s") * NC + lax.axis_index("c")

        @pl.when(wid < B)
        def _():
            b = wid
            pltpu.sync_copy(keys_hbm.at[b], keys_v)

            prefix = jnp.uint32(0)
            krem = jnp.int32(K)
            for p in range(4):
                shift = 24 - 8 * p
                prefmask = jnp.uint32((0xFFFFFFFF << (shift + 8)) & 0xFFFFFFFF)
                for j in range(16):
                    hist_v[pl.ds(j * L, L)] = jnp.zeros((L,), jnp.int32)

                ones = jnp.ones((L,), jnp.int32)

                def hist_body(i, _, prefix=prefix, shift=shift, prefmask=prefmask):
                    kv = keys_v[pl.ds(i * L, L)]
                    byte = ((kv >> jnp.uint32(shift)) & jnp.uint32(0xFF)).astype(
                        jnp.int32
                    )
                    if shift == 24:
                        plsc.addupdate_scatter(hist_v, [byte], ones)
                    else:
                        match = (kv & prefmask) == prefix
                        plsc.addupdate_scatter(hist_v, [byte], ones, mask=match)
                    return 0

                lax.fori_loop(0, NV, hist_body, 0)

                lane = lax.iota(jnp.int32, L)

                def scan_body(j, st, krem=krem):
                    carry, bbyte, krem_new = st
                    c = 15 - j
                    h = hist_v[pl.ds(c * L, L)]
                    srev = jnp.cumsum(jnp.flip(h))
                    s = jnp.flip(srev) + carry
                    tot = jnp.sum(h)
                    s0 = carry + tot
                    in_chunk = jnp.logical_and(carry < krem, s0 >= krem)
                    msk = s >= krem
                    l = jnp.sum(msk.astype(jnp.int32)) - 1
                    sl = jnp.sum(jnp.where(lane == l, s, 0))
                    hl = jnp.sum(jnp.where(lane == l, h, 0))
                    bbyte = jnp.where(in_chunk, c * L + l, bbyte)
                    krem_new = jnp.where(in_chunk, krem - (sl - hl), krem_new)
                    return (s0, bbyte, krem_new)

                _, bbyte, krem = lax.fori_loop(
                    0, 16, scan_body, (jnp.int32(0), jnp.int32(0), krem)
                )
                prefix = prefix | (bbyte.astype(jnp.uint32) << jnp.uint32(shift))

            thresh = prefix
            need = krem

            lane = lax.iota(jnp.int32, L)

            def comp_body(i, st):
                pos, tt = st
                kv = keys_v[pl.ds(i * L, L)]
                m_gt = kv > thresh
                m_eq = kv == thresh
                eqc = jnp.cumsum(m_eq.astype(jnp.int32))
                keep_eq = jnp.logical_and(m_eq, (tt + eqc) <= need)
                m = jnp.logical_or(m_gt, keep_eq)
                mc = jnp.cumsum(m.astype(jnp.int32))
                pos_v = jnp.clip(pos + mc - 1, 0, K - 1)
                plsc.store_scatter(idx_v, [pos_v], lane + i * L, mask=m)
                return (pos + jnp.sum(m.astype(jnp.int32)),
                        tt + jnp.sum(m_eq.astype(jnp.int32)))

            lax.fori_loop(0, NV, comp_body, (jnp.int32(0), jnp.int32(0)))
            pltpu.sync_copy(idx_v, idx_hbm.at[b])

    return select


def _make_gather(B, S, D, K):
    mesh = plsc.VectorSubcoreMesh(core_axis_name="c", subcore_axis_name="s")
    NC, NS = mesh.num_cores, mesh.num_subcores
    NW = NC * NS
    BK = B * K
    RPT = BK // NW
    CH = 16
    NCH = RPT // CH

    @functools.partial(
        pl.kernel,
        out_type=[
            jax.ShapeDtypeStruct((BK, D), jnp.float32),
            jax.ShapeDtypeStruct((BK,), jnp.int32),
        ],
        mesh=mesh,
        compiler_params=pltpu.CompilerParams(needs_layout_passes=False),
        scratch_types=[
            pltpu.VMEM((RPT,), jnp.int32),
            pltpu.VMEM((NCH, CH), jnp.int32),
            pltpu.VMEM((S,), jnp.int32),
            pltpu.VMEM((RPT,), jnp.int32),
            pltpu.VMEM((CH, D), jnp.float32),
            pltpu.VMEM((CH, D), jnp.float32),
            pltpu.SemaphoreType.DMA,
            pltpu.SemaphoreType.DMA,
        ],
    )
    def gather(hidden_hbm, mask_hbm, gidx_hbm, out_hbm, pmask_hbm,
               idx_v, idxc_v, mrow_v, pm_v, buf_a, buf_b, sem_a, sem_b):
        wid = lax.axis_index("s") * NC + lax.axis_index("c")
        base = wid * RPT
        b = base // K

        pltpu.sync_copy(gidx_hbm.at[pl.ds(base, RPT)], idx_v)
        pltpu.sync_copy(mask_hbm.at[b], mrow_v)

        boff = b * S

        def mg_body(j, _):
            iv = idx_v[pl.ds(j * L, L)]
            pm_v[pl.ds(j * L, L)] = plsc.load_gather(mrow_v, [iv])
            idxc_v[j] = iv + boff
            return 0

        lax.fori_loop(0, NCH, mg_body, 0)
        pltpu.sync_copy(pm_v, pmask_hbm.at[pl.ds(base, RPT)])

        def chunk_body(c, _):
            pltpu.async_copy(
                hidden_hbm.at[idxc_v.at[c]], buf_a, sem_a
            ).wait()
            pltpu.sync_copy(buf_a, out_hbm.at[pl.ds(base + c * CH, CH)])
            return 0

        lax.fori_loop(0, NCH, chunk_body, 0)

    return gather


def kernel(hidden_states, scores, attention_mask, keep_k):
    B, S, D = hidden_states.shape
    K = min(max(1, 4096), S - 1)

    bits = jax.lax.bitcast_convert_type(scores, jnp.int32)
    keys = jax.lax.bitcast_convert_type(bits, jnp.uint32) ^ (
        jax.lax.bitcast_convert_type(bits >> 31, jnp.uint32)
        | jnp.uint32(0x80000000)
    )
    idx = _make_select(B, S, K)(keys)

    off = jnp.clip(jnp.asarray(keep_k, jnp.int32), 1, S - 1) - jnp.int32(K)
    topk_indices = idx + off
    gidx = jnp.clip(topk_indices, 0, S - 1).reshape(B * K)

    hidden_flat = hidden_states.reshape(B * S, D)
    pruned_flat, pmask_flat = _make_gather(B, S, D, K)(
        hidden_flat, attention_mask, gidx
    )
    return (
        pruned_flat.reshape(B, K, D),
        pmask_flat.reshape(B, K),
        topk_indices,
    )


# speedup vs baseline: 1.5778x; 1.1096x over previous
import functools

import jax
import jax.numpy as jnp
from jax import lax
from jax.experimental import pallas as pl
from jax.experimental.pallas import tpu as pltpu
from jax.experimental.pallas import tpu_sc as plsc

L = 16


def _make_select(B, S, K):
    NV = S // L
    mesh = plsc.VectorSubcoreMesh(core_axis_name="c", subcore_axis_name="s")
    NC = mesh.num_cores

    @functools.partial(
        pl.kernel,
        out_type=jax.ShapeDtypeStruct((B, K), jnp.int32),
        mesh=mesh,
        compiler_params=pltpu.CompilerParams(needs_layout_passes=False),
        scratch_types=[
            pltpu.VMEM((S,), jnp.uint32),
            pltpu.VMEM((256,), jnp.int32),
            pltpu.VMEM((K,), jnp.int32),
        ],
    )
    def select(keys_hbm, idx_hbm, keys_v, hist_v, idx_v):
        wid = lax.axis_index("s") * NC + lax.axis_index("c")

        @pl.when(wid < B)
        def _():
            b = wid
            pltpu.sync_copy(keys_hbm.at[b], keys_v)

            prefix = jnp.uint32(0)
            krem = jnp.int32(K)
            for p in range(4):
                shift = 24 - 8 * p
                prefmask = jnp.uint32((0xFFFFFFFF << (shift + 8)) & 0xFFFFFFFF)
                for j in range(16):
                    hist_v[pl.ds(j * L, L)] = jnp.zeros((L,), jnp.int32)

                ones = jnp.ones((L,), jnp.int32)

                def hist_body(i, _, prefix=prefix, shift=shift, prefmask=prefmask):
                    kv = keys_v[pl.ds(i * L, L)]
                    byte = ((kv >> jnp.uint32(shift)) & jnp.uint32(0xFF)).astype(
                        jnp.int32
                    )
                    if shift == 24:
                        plsc.addupdate_scatter(hist_v, [byte], ones)
                    else:
                        match = (kv & prefmask) == prefix
                        plsc.addupdate_scatter(hist_v, [byte], ones, mask=match)
                    return 0

                lax.fori_loop(0, NV, hist_body, 0)

                lane = lax.iota(jnp.int32, L)

                def scan_body(j, st, krem=krem):
                    carry, bbyte, krem_new = st
                    c = 15 - j
                    h = hist_v[pl.ds(c * L, L)]
                    srev = jnp.cumsum(jnp.flip(h))
                    s = jnp.flip(srev) + carry
                    tot = jnp.sum(h)
                    s0 = carry + tot
                    in_chunk = jnp.logical_and(carry < krem, s0 >= krem)
                    msk = s >= krem
                    l = jnp.sum(msk.astype(jnp.int32)) - 1
                    sl = jnp.sum(jnp.where(lane == l, s, 0))
                    hl = jnp.sum(jnp.where(lane == l, h, 0))
                    bbyte = jnp.where(in_chunk, c * L + l, bbyte)
                    krem_new = jnp.where(in_chunk, krem - (sl - hl), krem_new)
                    return (s0, bbyte, krem_new)

                _, bbyte, krem = lax.fori_loop(
                    0, 16, scan_body, (jnp.int32(0), jnp.int32(0), krem)
                )
                prefix = prefix | (bbyte.astype(jnp.uint32) << jnp.uint32(shift))

            thresh = prefix
            need = krem

            lane = lax.iota(jnp.int32, L)

            def comp_body(i, st):
                pos, tt = st
                kv = keys_v[pl.ds(i * L, L)]
                m_gt = kv > thresh
                m_eq = kv == thresh
                eqc = jnp.cumsum(m_eq.astype(jnp.int32))
                keep_eq = jnp.logical_and(m_eq, (tt + eqc) <= need)
                m = jnp.logical_or(m_gt, keep_eq)
                mc = jnp.cumsum(m.astype(jnp.int32))
                pos_v = jnp.clip(pos + mc - 1, 0, K - 1)
                plsc.store_scatter(idx_v, [pos_v], lane + i * L, mask=m)
                return (pos + jnp.sum(m.astype(jnp.int32)),
                        tt + jnp.sum(m_eq.astype(jnp.int32)))

            lax.fori_loop(0, NV, comp_body, (jnp.int32(0), jnp.int32(0)))
            pltpu.sync_copy(idx_v, idx_hbm.at[b])

    return select


def _make_gather(B, S, D, K):
    mesh = plsc.VectorSubcoreMesh(core_axis_name="c", subcore_axis_name="s")
    NC, NS = mesh.num_cores, mesh.num_subcores
    NW = NC * NS
    BK = B * K
    RPT = BK // NW
    CH = 16
    NCH = RPT // CH

    @functools.partial(
        pl.kernel,
        out_type=[
            jax.ShapeDtypeStruct((BK, D), jnp.float32),
            jax.ShapeDtypeStruct((BK,), jnp.int32),
        ],
        mesh=mesh,
        compiler_params=pltpu.CompilerParams(needs_layout_passes=False),
        scratch_types=[
            pltpu.VMEM((RPT,), jnp.int32),
            pltpu.VMEM((NCH, CH), jnp.int32),
            pltpu.VMEM((S,), jnp.int32),
            pltpu.VMEM((RPT,), jnp.int32),
            pltpu.VMEM((CH, D), jnp.float32),
            pltpu.VMEM((CH, D), jnp.float32),
            pltpu.SemaphoreType.DMA,
            pltpu.SemaphoreType.DMA,
            pltpu.SemaphoreType.DMA,
            pltpu.SemaphoreType.DMA,
        ],
    )
    def gather(hidden_hbm, mask_hbm, gidx_hbm, out_hbm, pmask_hbm,
               idx_v, idxc_v, mrow_v, pm_v, buf_a, buf_b,
               gsem_a, gsem_b, wsem_a, wsem_b):
        wid = lax.axis_index("s") * NC + lax.axis_index("c")
        base = wid * RPT
        b = base // K

        pltpu.sync_copy(gidx_hbm.at[pl.ds(base, RPT)], idx_v)

        boff = b * S

        def idx_body(j, _):
            idxc_v[j] = idx_v[pl.ds(j * L, L)] + boff
            return 0

        lax.fori_loop(0, NCH, idx_body, 0)

        def start_g(c, buf, sem):
            pltpu.async_copy(hidden_hbm.at[idxc_v.at[c]], buf, sem)

        def wait_g(c, buf, sem):
            pltpu.make_async_copy(hidden_hbm.at[idxc_v.at[c]], buf, sem).wait()

        def start_w(c, buf, sem):
            pltpu.async_copy(buf, out_hbm.at[pl.ds(base + c * CH, CH)], sem)

        def wait_w(c, buf, sem):
            pltpu.make_async_copy(
                buf, out_hbm.at[pl.ds(base + c * CH, CH)], sem
            ).wait()

        start_g(0, buf_a, gsem_a)
        start_g(1, buf_b, gsem_b)

        pltpu.sync_copy(mask_hbm.at[b], mrow_v)

        def mg_body(j, _):
            iv = idx_v[pl.ds(j * L, L)]
            pm_v[pl.ds(j * L, L)] = plsc.load_gather(mrow_v, [iv])
            return 0

        lax.fori_loop(0, NCH, mg_body, 0)
        pltpu.sync_copy(pm_v, pmask_hbm.at[pl.ds(base, RPT)])

        wait_g(0, buf_a, gsem_a)
        start_w(0, buf_a, wsem_a)
        wait_g(1, buf_b, gsem_b)
        start_w(1, buf_b, wsem_b)

        def pair_body(g, _):
            c0 = 2 * g
            c1 = c0 + 1
            wait_w(c0 - 2, buf_a, wsem_a)
            start_g(c0, buf_a, gsem_a)
            wait_w(c1 - 2, buf_b, wsem_b)
            start_g(c1, buf_b, gsem_b)
            wait_g(c0, buf_a, gsem_a)
            start_w(c0, buf_a, wsem_a)
            wait_g(c1, buf_b, gsem_b)
            start_w(c1, buf_b, wsem_b)
            return 0

        lax.fori_loop(1, NCH // 2, pair_body, 0)
        wait_w(NCH - 2, buf_a, wsem_a)
        wait_w(NCH - 1, buf_b, wsem_b)

    return gather


def kernel(hidden_states, scores, attention_mask, keep_k):
    B, S, D = hidden_states.shape
    K = min(max(1, 4096), S - 1)

    bits = jax.lax.bitcast_convert_type(scores, jnp.int32)
    keys = jax.lax.bitcast_convert_type(bits, jnp.uint32) ^ (
        jax.lax.bitcast_convert_type(bits >> 31, jnp.uint32)
        | jnp.uint32(0x80000000)
    )
    idx = _make_select(B, S, K)(keys)

    off = jnp.clip(jnp.asarray(keep_k, jnp.int32), 1, S - 1) - jnp.int32(K)
    topk_indices = idx + off
    gidx = jnp.clip(topk_indices, 0, S - 1).reshape(B * K)

    hidden_flat = hidden_states.reshape(B * S, D)
    pruned_flat, pmask_flat = _make_gather(B, S, D, K)(
        hidden_flat, attention_mask, gidx
    )
    return (
        pruned_flat.reshape(B, K, D),
        pmask_flat.reshape(B, K),
        topk_indices,
    )


# speedup vs baseline: 1.6241x; 1.0294x over previous
import functools

import jax
import jax.numpy as jnp
from jax import lax
from jax.experimental import pallas as pl
from jax.experimental.pallas import tpu as pltpu
from jax.experimental.pallas import tpu_sc as plsc

L = 16


def _i32(x):
    x &= 0xFFFFFFFF
    return jnp.int32(x - (1 << 32) if x & 0x80000000 else x)


def _make_select(B, S, K):
    NV = S // L
    mesh = plsc.VectorSubcoreMesh(core_axis_name="c", subcore_axis_name="s")
    NC = mesh.num_cores

    @functools.partial(
        pl.kernel,
        out_type=jax.ShapeDtypeStruct((B, K), jnp.int32),
        mesh=mesh,
        compiler_params=pltpu.CompilerParams(needs_layout_passes=False),
        scratch_types=[
            pltpu.VMEM((S,), jnp.int32),
            pltpu.VMEM((256,), jnp.int32),
            pltpu.VMEM((S + L,), jnp.int32),
            pltpu.VMEM((K,), jnp.int32),
        ],
    )
    def select(keys_hbm, idx_hbm, keys_v, hist_v, cand_v, idx_v):
        wid = lax.axis_index("s") * NC + lax.axis_index("c")

        @pl.when(wid < B)
        def _():
            b = wid
            pltpu.sync_copy(keys_hbm.at[b], keys_v)

            lane = lax.iota(jnp.int32, L)
            ones = jnp.ones((L,), jnp.int32)

            def zero_hist():
                for j in range(16):
                    hist_v[pl.ds(j * L, L)] = jnp.zeros((L,), jnp.int32)

            def hist_scan(krem):
                def scan_body(j, st):
                    carry, dig, krem_new = st
                    c = 15 - j
                    h = hist_v[pl.ds(c * L, L)]
                    srev = jnp.cumsum(jnp.flip(h))
                    s = jnp.flip(srev) + carry
                    tot = jnp.sum(h)
                    s0 = carry + tot
                    in_chunk = jnp.logical_and(carry < krem, s0 >= krem)
                    msk = s >= krem
                    l = jnp.sum(msk.astype(jnp.int32)) - 1
                    sl = jnp.sum(jnp.where(lane == l, s, 0))
                    hl = jnp.sum(jnp.where(lane == l, h, 0))
                    dig = jnp.where(in_chunk, c * L + l, dig)
                    krem_new = jnp.where(in_chunk, krem - (sl - hl), krem_new)
                    return (s0, dig, krem_new)

                _, dig, krem = lax.fori_loop(
                    0, 16, scan_body, (jnp.int32(0), jnp.int32(0), krem)
                )
                return dig, krem

            zero_hist()

            def hist1_body(i, _):
                kv = keys_v[pl.ds(i * L, L)]
                dig = ((kv >> 24) & 0xFF) ^ 0x80
                plsc.addupdate_scatter(hist_v, [dig], ones)
                return 0

            lax.fori_loop(0, NV, hist1_body, 0)
            d1, krem = hist_scan(jnp.int32(K))
            rawb = d1 ^ 0x80
            prefix = rawb << 24

            def cand_body(i, st):
                pos = st
                kv = keys_v[pl.ds(i * L, L)]
                m = ((kv >> 24) & 0xFF) == rawb
                mc = jnp.cumsum(m.astype(jnp.int32))
                plsc.store_scatter(
                    cand_v, [jnp.maximum(pos + mc - 1, 0)], kv, mask=m
                )
                return pos + jnp.sum(m.astype(jnp.int32))

            ncand = lax.fori_loop(0, NV, cand_body, jnp.int32(0))
            pad = jnp.full((L,), 0, jnp.int32) + (prefix ^ _i32(0xFF000000))
            plsc.store_scatter(cand_v, [ncand + lane], pad)
            ncv = (ncand + (L - 1)) // L

            for p in range(1, 4):
                shift = 24 - 8 * p
                prefmask = _i32(0xFFFFFFFF << (shift + 8))
                zero_hist()

                def histp_body(i, _, shift=shift, prefmask=prefmask,
                               prefix=prefix):
                    kv = cand_v[pl.ds(i * L, L)]
                    match = (kv & prefmask) == prefix
                    dig = (kv >> shift) & 0xFF
                    plsc.addupdate_scatter(hist_v, [dig], ones, mask=match)
                    return 0

                lax.fori_loop(0, ncv, histp_body, 0)
                d, krem = hist_scan(krem)
                prefix = prefix | (d << shift)

            thresh = prefix
            need = krem

            lane = lax.iota(jnp.int32, L)

            def comp_body(i, st):
                pos, tt = st
                kv = keys_v[pl.ds(i * L, L)]
                m_gt = kv > thresh
                m_eq = kv == thresh
                eqc = jnp.cumsum(m_eq.astype(jnp.int32))
                keep_eq = jnp.logical_and(m_eq, (tt + eqc) <= need)
                m = jnp.logical_or(m_gt, keep_eq)
                mc = jnp.cumsum(m.astype(jnp.int32))
                pos_v = jnp.clip(pos + mc - 1, 0, K - 1)
                plsc.store_scatter(idx_v, [pos_v], lane + i * L, mask=m)
                return (pos + jnp.sum(m.astype(jnp.int32)),
                        tt + jnp.sum(m_eq.astype(jnp.int32)))

            lax.fori_loop(0, NV, comp_body, (jnp.int32(0), jnp.int32(0)))
            pltpu.sync_copy(idx_v, idx_hbm.at[b])

    return select


def _make_gather(B, S, D, K):
    mesh = plsc.VectorSubcoreMesh(core_axis_name="c", subcore_axis_name="s")
    NC, NS = mesh.num_cores, mesh.num_subcores
    NW = NC * NS
    BK = B * K
    RPT = BK // NW
    CH = 16
    NCH = RPT // CH

    @functools.partial(
        pl.kernel,
        out_type=[
            jax.ShapeDtypeStruct((BK, D), jnp.float32),
            jax.ShapeDtypeStruct((BK,), jnp.int32),
        ],
        mesh=mesh,
        compiler_params=pltpu.CompilerParams(needs_layout_passes=False),
        scratch_types=[
            pltpu.VMEM((RPT,), jnp.int32),
            pltpu.VMEM((NCH, CH), jnp.int32),
            pltpu.VMEM((S,), jnp.int32),
            pltpu.VMEM((RPT,), jnp.int32),
            pltpu.VMEM((CH, D), jnp.float32),
            pltpu.VMEM((CH, D), jnp.float32),
            pltpu.SemaphoreType.DMA,
            pltpu.SemaphoreType.DMA,
            pltpu.SemaphoreType.DMA,
            pltpu.SemaphoreType.DMA,
        ],
    )
    def gather(hidden_hbm, mask_hbm, gidx_hbm, out_hbm, pmask_hbm,
               idx_v, idxc_v, mrow_v, pm_v, buf_a, buf_b,
               gsem_a, gsem_b, wsem_a, wsem_b):
        wid = lax.axis_index("s") * NC + lax.axis_index("c")
        base = wid * RPT
        b = base // K

        pltpu.sync_copy(gidx_hbm.at[pl.ds(base, RPT)], idx_v)

        boff = b * S

        def idx_body(j, _):
            idxc_v[j] = idx_v[pl.ds(j * L, L)] + boff
            return 0

        lax.fori_loop(0, NCH, idx_body, 0)

        def start_g(c, buf, sem):
            pltpu.async_copy(hidden_hbm.at[idxc_v.at[c]], buf, sem)

        def wait_g(c, buf, sem):
            pltpu.make_async_copy(hidden_hbm.at[idxc_v.at[c]], buf, sem).wait()

        def start_w(c, buf, sem):
            pltpu.async_copy(buf, out_hbm.at[pl.ds(base + c * CH, CH)], sem)

        def wait_w(c, buf, sem):
            pltpu.make_async_copy(
                buf, out_hbm.at[pl.ds(base + c * CH, CH)], sem
            ).wait()

        start_g(0, buf_a, gsem_a)
        start_g(1, buf_b, gsem_b)

        pltpu.sync_copy(mask_hbm.at[b], mrow_v)

        def mg_body(j, _):
            iv = idx_v[pl.ds(j * L, L)]
            pm_v[pl.ds(j * L, L)] = plsc.load_gather(mrow_v, [iv])
            return 0

        lax.fori_loop(0, NCH, mg_body, 0)
        pltpu.sync_copy(pm_v, pmask_hbm.at[pl.ds(base, RPT)])

        wait_g(0, buf_a, gsem_a)
        start_w(0, buf_a, wsem_a)
        wait_g(1, buf_b, gsem_b)
        start_w(1, buf_b, wsem_b)

        def pair_body(g, _):
            c0 = 2 * g
            c1 = c0 + 1
            wait_w(c0 - 2, buf_a, wsem_a)
            start_g(c0, buf_a, gsem_a)
            wait_w(c1 - 2, buf_b, wsem_b)
            start_g(c1, buf_b, gsem_b)
            wait_g(c0, buf_a, gsem_a)
            start_w(c0, buf_a, wsem_a)
            wait_g(c1, buf_b, gsem_b)
            start_w(c1, buf_b, wsem_b)
            return 0

        lax.fori_loop(1, NCH // 2, pair_body, 0)
        wait_w(NCH - 2, buf_a, wsem_a)
        wait_w(NCH - 1, buf_b, wsem_b)

    return gather


def kernel(hidden_states, scores, attention_mask, keep_k):
    B, S, D = hidden_states.shape
    K = min(max(1, 4096), S - 1)

    bits = jax.lax.bitcast_convert_type(scores, jnp.int32)
    keys = bits ^ ((bits >> 31) & jnp.int32(0x7FFFFFFF))
    idx = _make_select(B, S, K)(keys)

    off = jnp.clip(jnp.asarray(keep_k, jnp.int32), 1, S - 1) - jnp.int32(K)
    topk_indices = idx + off
    gidx = jnp.clip(topk_indices, 0, S - 1).reshape(B * K)

    hidden_flat = hidden_states.reshape(B * S, D)
    pruned_flat, pmask_flat = _make_gather(B, S, D, K)(
        hidden_flat, attention_mask, gidx
    )
    return (
        pruned_flat.reshape(B, K, D),
        pmask_flat.reshape(B, K),
        topk_indices,
    )
